# Initial kernel scaffold; baseline (speedup 1.0000x reference)
import jax
import jax.numpy as jnp
from jax.experimental import pallas as pl


def kernel(x, edge_index, br_r, br_x, g_fr, b_fr, g_to, b_to, tap, shift, p_spec, q_spec, gs, bs, bus_type, vm_setpoint):
    raise NotImplementedError("write your pallas kernel here")


# speedup vs baseline: 128.0489x; 128.0489x over previous
import functools

import jax
import jax.numpy as jnp
from jax import lax
from jax.experimental import pallas as pl
from jax.experimental.pallas import tpu as pltpu
from jax.experimental.pallas import tpu_sc as plsc

N_CORES = 2
N_SUBCORES = 16
LANES = 16
N_TILES = N_CORES * N_SUBCORES

ROW_W = 80
ROWS_PER_CHUNK = 40
CHUNK = ROWS_PER_CHUNK * ROW_W

TWO_OVER_PI = 0.6366197723675814
PIO2 = 1.5707963267948966
MAGIC = 12582912.0
S1, S2, S3 = -1.6666654611e-1, 8.3321608736e-3, -1.9515295891e-4
C1, C2, C3 = 4.166664568298827e-2, -1.388731625493765e-3, 2.443315711809948e-5


def _sincos(ang):
    jf = ang * TWO_OVER_PI
    jr = (jf + MAGIC) - MAGIC
    y = ang - jr * PIO2
    qi = jr.astype(jnp.int32)
    z = y * y
    sinp = ((S3 * z + S2) * z + S1) * z * y + y
    cosp = ((C3 * z + C2) * z + C1) * (z * z) - 0.5 * z + 1.0
    swap = (qi & 1) == 1
    nsin = (qi & 2) != 0
    ncos = ((qi + 1) & 2) != 0
    s = jnp.where(swap, cosp, sinp)
    s = jnp.where(nsin, -s, s)
    c = jnp.where(swap, sinp, cosp)
    c = jnp.where(ncos, -c, c)
    return s, c


def _make_sc_kernel(n, n_pad, n_edges):
    n_chunks_total = n_edges // CHUNK
    base_chunks = n_chunks_total // N_TILES
    extra = n_chunks_total - base_chunks * N_TILES
    npt = n_pad // N_SUBCORES

    mesh = plsc.VectorSubcoreMesh(core_axis_name="c", subcore_axis_name="s")

    @functools.partial(
        pl.kernel,
        out_type=(
            jax.ShapeDtypeStruct((N_CORES, 1, n_pad), jnp.float32),
            jax.ShapeDtypeStruct((N_CORES, 1, n_pad), jnp.float32),
        ),
        mesh=mesh,
        scratch_types=[
            pltpu.VMEM_SHARED((n_pad,), jnp.float32),
            pltpu.VMEM_SHARED((n_pad,), jnp.float32),
            pltpu.VMEM_SHARED((n_pad,), jnp.float32),
            pltpu.VMEM_SHARED((n_pad,), jnp.float32),
            pltpu.VMEM((ROWS_PER_CHUNK, ROW_W), jnp.int32),
            pltpu.VMEM((ROWS_PER_CHUNK, ROW_W), jnp.int32),
            pltpu.VMEM((8, CHUNK), jnp.float32),
            pltpu.VMEM((CHUNK,), jnp.float32),
            pltpu.VMEM((CHUNK,), jnp.float32),
            pltpu.VMEM((CHUNK,), jnp.float32),
            pltpu.VMEM((CHUNK,), jnp.float32),
            pltpu.VMEM((CHUNK,), jnp.float32),
            pltpu.VMEM((CHUNK,), jnp.float32),
            pltpu.VMEM((CHUNK,), jnp.float32),
            pltpu.VMEM((CHUNK,), jnp.float32),
            pltpu.VMEM((n_pad // N_SUBCORES,), jnp.float32),
            pltpu.SemaphoreType.DMA,
            pltpu.SemaphoreType.DMA,
            pltpu.SemaphoreType.DMA,
        ],
    )
    def sc_kernel(x_hbm, ei_hbm, r_hbm, xx_hbm, gfr_hbm, bfr_hbm,
                  gto_hbm, bto_hbm, tap_hbm, shift_hbm, outp_hbm, outq_hbm,
                  va_sh, vm_sh, accp_sh, accq_sh, si, di, att,
                  g_vai, g_vmi, g_vaj, g_vmj, c_pf, c_qf, c_pt, c_qt,
                  vbuf, sem_in, sem_g, sem_s):
        c = lax.axis_index("c")
        s = lax.axis_index("s")
        wid = c * N_SUBCORES + s
        npt_ = n_pad // N_SUBCORES
        last = n - (N_SUBCORES - 1) * npt_

        fz16 = jnp.zeros((LANES,), jnp.float32)

        @pl.loop(0, npt_ // LANES)
        def _z(v):
            vbuf[pl.ds(v * LANES, LANES)] = fz16

        tsl = pl.ds(s * npt_, npt_)
        pltpu.sync_copy(vbuf, accp_sh.at[tsl])
        pltpu.sync_copy(vbuf, accq_sh.at[tsl])

        @pl.when(s < N_SUBCORES - 1)
        def _():
            pltpu.sync_copy(x_hbm.at[pl.ds(s * npt_, npt_)], vbuf)
            pltpu.sync_copy(vbuf, va_sh.at[tsl])
            pltpu.sync_copy(x_hbm.at[pl.ds(n + s * npt_, npt_)], vbuf)
            pltpu.sync_copy(vbuf, vm_sh.at[tsl])

        @pl.when(s == N_SUBCORES - 1)
        def _():
            lsl = pl.ds(0, last)
            ssl = pl.ds(s * npt_, last)
            pltpu.sync_copy(x_hbm.at[pl.ds(s * npt_, last)], vbuf.at[lsl])
            pltpu.sync_copy(vbuf.at[lsl], va_sh.at[ssl])
            pltpu.sync_copy(x_hbm.at[pl.ds(n + s * npt_, last)], vbuf.at[lsl])
            pltpu.sync_copy(vbuf.at[lsl], vm_sh.at[ssl])

        plsc.subcore_barrier()

        attr_hbm = (r_hbm, xx_hbm, gfr_hbm, bfr_hbm, gto_hbm, bto_hbm,
                    tap_hbm, shift_hbm)
        my_chunks = jnp.where(wid < extra, base_chunks + 1, base_chunks)

        @pl.loop(0, my_chunks)
        def _chunk(t):
            ki = wid + N_TILES * t
            eoff = ki * CHUNK
            roff = ki * ROWS_PER_CHUNK
            cps = [
                pltpu.async_copy(ei_hbm.at[0, pl.ds(roff, ROWS_PER_CHUNK)],
                                 si, sem_in),
                pltpu.async_copy(ei_hbm.at[1, pl.ds(roff, ROWS_PER_CHUNK)],
                                 di, sem_in),
            ]
            for m, h in enumerate(attr_hbm):
                cps.append(pltpu.async_copy(h.at[pl.ds(eoff, CHUNK)],
                                            att.at[m], sem_in))
            for cp in cps:
                cp.wait()

            @pl.loop(0, ROWS_PER_CHUNK)
            def _g(j):
                dsl = pl.ds(j * ROW_W, ROW_W)
                pltpu.async_copy(va_sh.at[si.at[j]], g_vai.at[dsl], sem_g)
                pltpu.async_copy(vm_sh.at[si.at[j]], g_vmi.at[dsl], sem_g)
                pltpu.async_copy(va_sh.at[di.at[j]], g_vaj.at[dsl], sem_g)
                pltpu.async_copy(vm_sh.at[di.at[j]], g_vmj.at[dsl], sem_g)

            @pl.loop(0, ROWS_PER_CHUNK)
            def _gw(j):
                dsl = pl.ds(j * ROW_W, ROW_W)
                pltpu.make_async_copy(va_sh.at[si.at[j]], g_vai.at[dsl],
                                      sem_g).wait()
                pltpu.make_async_copy(vm_sh.at[si.at[j]], g_vmi.at[dsl],
                                      sem_g).wait()
                pltpu.make_async_copy(va_sh.at[di.at[j]], g_vaj.at[dsl],
                                      sem_g).wait()
                pltpu.make_async_copy(vm_sh.at[di.at[j]], g_vmj.at[dsl],
                                      sem_g).wait()

            @pl.loop(0, CHUNK // LANES)
            def _compute(v):
                sl = pl.ds(v * LANES, LANES)
                va_i = g_vai[sl]
                vm_i = g_vmi[sl]
                va_j = g_vaj[sl]
                vm_j = g_vmj[sl]
                br_r = att.at[0][sl]
                br_x = att.at[1][sl]
                gfr = att.at[2][sl]
                bfr = att.at[3][sl]
                gto = att.at[4][sl]
                bto = att.at[5][sl]
                tp = att.at[6][sl]
                sh = att.at[7][sl]

                inv_den = 1.0 / (br_r * br_r + br_x * br_x)
                g_s = br_r * inv_den
                b_s = -br_x * inv_den
                inv_tap = 1.0 / tp
                vit = vm_i * inv_tap
                vit2 = vit * vit
                vij = vit * vm_j
                sin_f, cos_f = _sincos(va_i - va_j - sh)
                gc = g_s * cos_f
                gs_ = g_s * sin_f
                bc = b_s * cos_f
                bs_ = b_s * sin_f
                c_pf[sl] = vit2 * (g_s + gfr) + vij * (-gc - bs_)
                c_qf[sl] = -vit2 * (b_s + bfr) + vij * (-gs_ + bc)
                vj2 = vm_j * vm_j
                c_pt[sl] = vj2 * (g_s + gto) + vij * (-gc + bs_)
                c_qt[sl] = -vj2 * (b_s + bto) + vij * (gs_ + bc)

            @pl.loop(0, ROWS_PER_CHUNK)
            def _sc(j):
                dsl = pl.ds(j * ROW_W, ROW_W)
                pltpu.async_copy(c_pf.at[dsl], accp_sh.at[si.at[j]], sem_s,
                                 add=True)
                pltpu.async_copy(c_qf.at[dsl], accq_sh.at[si.at[j]], sem_s,
                                 add=True)
                pltpu.async_copy(c_pt.at[dsl], accp_sh.at[di.at[j]], sem_s,
                                 add=True)
                pltpu.async_copy(c_qt.at[dsl], accq_sh.at[di.at[j]], sem_s,
                                 add=True)

            @pl.loop(0, ROWS_PER_CHUNK)
            def _scw(j):
                dsl = pl.ds(j * ROW_W, ROW_W)
                pltpu.make_async_copy(c_pf.at[dsl], accp_sh.at[si.at[j]],
                                      sem_s).wait()
                pltpu.make_async_copy(c_qf.at[dsl], accq_sh.at[si.at[j]],
                                      sem_s).wait()
                pltpu.make_async_copy(c_pt.at[dsl], accp_sh.at[di.at[j]],
                                      sem_s).wait()
                pltpu.make_async_copy(c_qt.at[dsl], accq_sh.at[di.at[j]],
                                      sem_s).wait()

        plsc.subcore_barrier()

        dsl = pl.ds(s * npt, npt)
        pltpu.sync_copy(accp_sh.at[dsl], vbuf)
        pltpu.sync_copy(vbuf, outp_hbm.at[c, 0, dsl])
        pltpu.sync_copy(accq_sh.at[dsl], vbuf)
        pltpu.sync_copy(vbuf, outq_hbm.at[c, 0, dsl])

    return sc_kernel


def _finalize_body(pp_ref, qq_ref, x_ref, ps_ref, qs_ref, gs_ref, bs_ref,
                   bt_ref, vs_ref, o_ref):
    va = x_ref[0, :]
    vm = x_ref[1, :]
    vm2 = vm * vm
    p_calc = pp_ref[0, :] + pp_ref[1, :] + vm2 * gs_ref[...]
    q_calc = qq_ref[0, :] + qq_ref[1, :] - vm2 * bs_ref[...]
    bt = bt_ref[...]
    sl_mask = bt == 3
    pvsl = (bt == 2) | sl_mask
    o_ref[0, :] = jnp.where(sl_mask, va, ps_ref[...] - p_calc)
    o_ref[1, :] = jnp.where(pvsl, vm - vs_ref[...], qs_ref[...] - q_calc)


def kernel(x, edge_index, br_r, br_x, g_fr, b_fr, g_to, b_to, tap, shift,
           p_spec, q_spec, gs, bs, bus_type, vm_setpoint):
    n = x.shape[0] // 2
    e = edge_index.shape[1]
    group = N_SUBCORES * 128
    n_pad = ((n + group - 1) // group) * group
    ei3 = edge_index.reshape(2, e // ROW_W, ROW_W)

    partp, partq = _make_sc_kernel(n, n_pad, e)(
        x, ei3, br_r, br_x, g_fr, b_fr, g_to, b_to, tap, shift)
    pp = partp.reshape(2, n_pad)[:, :n]
    qq = partq.reshape(2, n_pad)[:, :n]

    out2 = pl.pallas_call(
        _finalize_body,
        out_shape=jax.ShapeDtypeStruct((2, n), jnp.float32),
    )(pp, qq, x.reshape(2, n), p_spec, q_spec, gs, bs, bus_type, vm_setpoint)
    return out2.reshape(2 * n)


# speedup vs baseline: 151.4807x; 1.1830x over previous
import functools

import jax
import jax.numpy as jnp
from jax import lax
from jax.experimental import pallas as pl
from jax.experimental.pallas import tpu as pltpu
from jax.experimental.pallas import tpu_sc as plsc

N_CORES = 2
N_SUBCORES = 16
LANES = 16
N_TILES = N_CORES * N_SUBCORES

ROW_W = 80
ROWS_PER_CHUNK = 40
CHUNK = ROWS_PER_CHUNK * ROW_W

TWO_OVER_PI = 0.6366197723675814
PIO2 = 1.5707963267948966
MAGIC = 12582912.0
S1, S2, S3 = -1.6666654611e-1, 8.3321608736e-3, -1.9515295891e-4
C1, C2, C3 = 4.166664568298827e-2, -1.388731625493765e-3, 2.443315711809948e-5


def _sincos(ang):
    jf = ang * TWO_OVER_PI
    jr = (jf + MAGIC) - MAGIC
    y = ang - jr * PIO2
    qi = jr.astype(jnp.int32)
    z = y * y
    sinp = ((S3 * z + S2) * z + S1) * z * y + y
    cosp = ((C3 * z + C2) * z + C1) * (z * z) - 0.5 * z + 1.0
    swap = (qi & 1) == 1
    nsin = (qi & 2) != 0
    ncos = ((qi + 1) & 2) != 0
    s = jnp.where(swap, cosp, sinp)
    s = jnp.where(nsin, -s, s)
    c = jnp.where(swap, sinp, cosp)
    c = jnp.where(ncos, -c, c)
    return s, c


def _make_sc_kernel(n, n_pad, n_edges):
    n_chunks_total = n_edges // CHUNK
    base_chunks = n_chunks_total // N_TILES
    extra = n_chunks_total - base_chunks * N_TILES
    npt = n_pad // N_SUBCORES

    mesh = plsc.VectorSubcoreMesh(core_axis_name="c", subcore_axis_name="s")

    @functools.partial(
        pl.kernel,
        out_type=(
            jax.ShapeDtypeStruct((N_CORES, 1, n_pad), jnp.float32),
            jax.ShapeDtypeStruct((N_CORES, 1, n_pad), jnp.float32),
        ),
        mesh=mesh,
        scratch_types=[
            pltpu.VMEM_SHARED((n_pad,), jnp.float32),
            pltpu.VMEM_SHARED((n_pad,), jnp.float32),
            pltpu.VMEM_SHARED((n_pad,), jnp.float32),
            pltpu.VMEM_SHARED((n_pad,), jnp.float32),
            pltpu.VMEM((ROWS_PER_CHUNK, ROW_W), jnp.int32),
            pltpu.VMEM((ROWS_PER_CHUNK, ROW_W), jnp.int32),
            pltpu.VMEM((8, CHUNK), jnp.float32),
            pltpu.VMEM((CHUNK,), jnp.float32),
            pltpu.VMEM((CHUNK,), jnp.float32),
            pltpu.VMEM((CHUNK,), jnp.float32),
            pltpu.VMEM((CHUNK,), jnp.float32),
            pltpu.VMEM((CHUNK,), jnp.float32),
            pltpu.VMEM((CHUNK,), jnp.float32),
            pltpu.VMEM((CHUNK,), jnp.float32),
            pltpu.VMEM((CHUNK,), jnp.float32),
            pltpu.VMEM((n_pad // N_SUBCORES,), jnp.float32),
            pltpu.SemaphoreType.DMA,
            pltpu.SemaphoreType.DMA,
            pltpu.SemaphoreType.DMA,
            pltpu.SemaphoreType.DMA,
        ],
    )
    def sc_kernel(x_hbm, ei_hbm, r_hbm, xx_hbm, gfr_hbm, bfr_hbm,
                  gto_hbm, bto_hbm, tap_hbm, shift_hbm, outp_hbm, outq_hbm,
                  va_sh, vm_sh, accp_sh, accq_sh, si, di, att,
                  g_vai, g_vmi, g_vaj, g_vmj, c_pf, c_qf, c_pt, c_qt,
                  vbuf, sem_in, sem_ga, sem_gb, sem_s):
        c = lax.axis_index("c")
        s = lax.axis_index("s")
        wid = s * N_CORES + c
        npt_ = n_pad // N_SUBCORES
        last = n - (N_SUBCORES - 1) * npt_

        fz16 = jnp.zeros((LANES,), jnp.float32)

        @pl.loop(0, npt_ // LANES)
        def _z(v):
            vbuf[pl.ds(v * LANES, LANES)] = fz16

        tsl = pl.ds(s * npt_, npt_)
        pltpu.sync_copy(vbuf, accp_sh.at[tsl])
        pltpu.sync_copy(vbuf, accq_sh.at[tsl])

        @pl.when(s < N_SUBCORES - 1)
        def _():
            pltpu.sync_copy(x_hbm.at[pl.ds(s * npt_, npt_)], vbuf)
            pltpu.sync_copy(vbuf, va_sh.at[tsl])
            pltpu.sync_copy(x_hbm.at[pl.ds(n + s * npt_, npt_)], vbuf)
            pltpu.sync_copy(vbuf, vm_sh.at[tsl])

        @pl.when(s == N_SUBCORES - 1)
        def _():
            lsl = pl.ds(0, last)
            ssl = pl.ds(s * npt_, last)
            pltpu.sync_copy(x_hbm.at[pl.ds(s * npt_, last)], vbuf.at[lsl])
            pltpu.sync_copy(vbuf.at[lsl], va_sh.at[ssl])
            pltpu.sync_copy(x_hbm.at[pl.ds(n + s * npt_, last)], vbuf.at[lsl])
            pltpu.sync_copy(vbuf.at[lsl], vm_sh.at[ssl])

        plsc.subcore_barrier()

        attr_hbm = (r_hbm, xx_hbm, gfr_hbm, bfr_hbm, gto_hbm, bto_hbm,
                    tap_hbm, shift_hbm)
        my_chunks = jnp.where(wid < extra, base_chunks + 1, base_chunks)

        @pl.loop(0, my_chunks)
        def _chunk(t):
            ki = wid + N_TILES * t
            eoff = ki * CHUNK
            roff = ki * ROWS_PER_CHUNK
            cps = [
                pltpu.async_copy(ei_hbm.at[0, pl.ds(roff, ROWS_PER_CHUNK)],
                                 si, sem_in),
                pltpu.async_copy(ei_hbm.at[1, pl.ds(roff, ROWS_PER_CHUNK)],
                                 di, sem_in),
            ]
            for m, h in enumerate(attr_hbm):
                cps.append(pltpu.async_copy(h.at[pl.ds(eoff, CHUNK)],
                                            att.at[m], sem_in))
            for cp in cps:
                cp.wait()

            def fire_row(j, sem):
                dsl = pl.ds(j * ROW_W, ROW_W)
                pltpu.async_copy(va_sh.at[si.at[j]], g_vai.at[dsl], sem)
                pltpu.async_copy(vm_sh.at[si.at[j]], g_vmi.at[dsl], sem)
                pltpu.async_copy(va_sh.at[di.at[j]], g_vaj.at[dsl], sem)
                pltpu.async_copy(vm_sh.at[di.at[j]], g_vmj.at[dsl], sem)

            def wait_row(j, sem):
                dsl = pl.ds(j * ROW_W, ROW_W)
                pltpu.make_async_copy(va_sh.at[si.at[j]], g_vai.at[dsl],
                                      sem).wait()
                pltpu.make_async_copy(vm_sh.at[si.at[j]], g_vmi.at[dsl],
                                      sem).wait()
                pltpu.make_async_copy(va_sh.at[di.at[j]], g_vaj.at[dsl],
                                      sem).wait()
                pltpu.make_async_copy(vm_sh.at[di.at[j]], g_vmj.at[dsl],
                                      sem).wait()

            def fire_scat(j):
                dsl = pl.ds(j * ROW_W, ROW_W)
                pltpu.async_copy(c_pf.at[dsl], accp_sh.at[si.at[j]], sem_s,
                                 add=True)
                pltpu.async_copy(c_qf.at[dsl], accq_sh.at[si.at[j]], sem_s,
                                 add=True)
                pltpu.async_copy(c_pt.at[dsl], accp_sh.at[di.at[j]], sem_s,
                                 add=True)
                pltpu.async_copy(c_qt.at[dsl], accq_sh.at[di.at[j]], sem_s,
                                 add=True)

            fire_row(0, sem_ga)

            @pl.loop(0, ROWS_PER_CHUNK)
            def _row(j):
                even = (j & 1) == 0

                @pl.when(even)
                def _():
                    @pl.when(j + 1 < ROWS_PER_CHUNK)
                    def _():
                        fire_row(j + 1, sem_gb)
                    wait_row(j, sem_ga)

                @pl.when(jnp.logical_not(even))
                def _():
                    @pl.when(j + 1 < ROWS_PER_CHUNK)
                    def _():
                        fire_row(j + 1, sem_ga)
                    wait_row(j, sem_gb)

                for v in range(ROW_W // LANES):
                    sl = pl.ds(j * ROW_W + v * LANES, LANES)
                    va_i = g_vai[sl]
                    vm_i = g_vmi[sl]
                    va_j = g_vaj[sl]
                    vm_j = g_vmj[sl]
                    br_r = att.at[0][sl]
                    br_x = att.at[1][sl]
                    gfr = att.at[2][sl]
                    bfr = att.at[3][sl]
                    gto = att.at[4][sl]
                    bto = att.at[5][sl]
                    tp = att.at[6][sl]
                    sh = att.at[7][sl]

                    inv_den = 1.0 / (br_r * br_r + br_x * br_x)
                    g_s = br_r * inv_den
                    b_s = -br_x * inv_den
                    inv_tap = 1.0 / tp
                    vit = vm_i * inv_tap
                    vit2 = vit * vit
                    vij = vit * vm_j
                    sin_f, cos_f = _sincos(va_i - va_j - sh)
                    gc = g_s * cos_f
                    gs_ = g_s * sin_f
                    bc = b_s * cos_f
                    bs_ = b_s * sin_f
                    c_pf[sl] = vit2 * (g_s + gfr) + vij * (-gc - bs_)
                    c_qf[sl] = -vit2 * (b_s + bfr) + vij * (-gs_ + bc)
                    vj2 = vm_j * vm_j
                    c_pt[sl] = vj2 * (g_s + gto) + vij * (-gc + bs_)
                    c_qt[sl] = -vj2 * (b_s + bto) + vij * (gs_ + bc)

                @pl.when(j > 0)
                def _():
                    fire_scat(j - 1)

            fire_scat(ROWS_PER_CHUNK - 1)

            @pl.loop(0, ROWS_PER_CHUNK)
            def _scw(j):
                dsl = pl.ds(j * ROW_W, ROW_W)
                pltpu.make_async_copy(c_pf.at[dsl], accp_sh.at[si.at[j]],
                                      sem_s).wait()
                pltpu.make_async_copy(c_qf.at[dsl], accq_sh.at[si.at[j]],
                                      sem_s).wait()
                pltpu.make_async_copy(c_pt.at[dsl], accp_sh.at[di.at[j]],
                                      sem_s).wait()
                pltpu.make_async_copy(c_qt.at[dsl], accq_sh.at[di.at[j]],
                                      sem_s).wait()

        plsc.subcore_barrier()

        dsl = pl.ds(s * npt, npt)
        pltpu.sync_copy(accp_sh.at[dsl], vbuf)
        pltpu.sync_copy(vbuf, outp_hbm.at[c, 0, dsl])
        pltpu.sync_copy(accq_sh.at[dsl], vbuf)
        pltpu.sync_copy(vbuf, outq_hbm.at[c, 0, dsl])

    return sc_kernel


def _finalize_body(pp_ref, qq_ref, x_ref, ps_ref, qs_ref, gs_ref, bs_ref,
                   bt_ref, vs_ref, o_ref):
    va = x_ref[0, :]
    vm = x_ref[1, :]
    vm2 = vm * vm
    p_calc = pp_ref[0, :] + pp_ref[1, :] + vm2 * gs_ref[...]
    q_calc = qq_ref[0, :] + qq_ref[1, :] - vm2 * bs_ref[...]
    bt = bt_ref[...]
    sl_mask = bt == 3
    pvsl = (bt == 2) | sl_mask
    o_ref[0, :] = jnp.where(sl_mask, va, ps_ref[...] - p_calc)
    o_ref[1, :] = jnp.where(pvsl, vm - vs_ref[...], qs_ref[...] - q_calc)


def kernel(x, edge_index, br_r, br_x, g_fr, b_fr, g_to, b_to, tap, shift,
           p_spec, q_spec, gs, bs, bus_type, vm_setpoint):
    n = x.shape[0] // 2
    e = edge_index.shape[1]
    group = N_SUBCORES * 128
    n_pad = ((n + group - 1) // group) * group
    ei3 = edge_index.reshape(2, e // ROW_W, ROW_W)

    partp, partq = _make_sc_kernel(n, n_pad, e)(
        x, ei3, br_r, br_x, g_fr, b_fr, g_to, b_to, tap, shift)
    pp = partp.reshape(2, n_pad)[:, :n]
    qq = partq.reshape(2, n_pad)[:, :n]

    out2 = pl.pallas_call(
        _finalize_body,
        out_shape=jax.ShapeDtypeStruct((2, n), jnp.float32),
    )(pp, qq, x.reshape(2, n), p_spec, q_spec, gs, bs, bus_type, vm_setpoint)
    return out2.reshape(2 * n)


# speedup vs baseline: 332.3062x; 2.1937x over previous
import functools

import jax
import jax.numpy as jnp
from jax import lax
from jax.experimental import pallas as pl
from jax.experimental.pallas import tpu as pltpu
from jax.experimental.pallas import tpu_sc as plsc

N_CORES = 2
N_SUBCORES = 16
LANES = 16
N_TILES = N_CORES * N_SUBCORES

ROW_W = 128
ROWS_PER_CHUNK = 40
CHUNK = ROWS_PER_CHUNK * ROW_W

TWO_OVER_PI = 0.6366197723675814
PIO2 = 1.5707963267948966
MAGIC = 12582912.0
S1, S2, S3 = -1.6666654611e-1, 8.3321608736e-3, -1.9515295891e-4
C1, C2, C3 = 4.166664568298827e-2, -1.388731625493765e-3, 2.443315711809948e-5


def _sincos(ang):
    jf = ang * TWO_OVER_PI
    jr = (jf + MAGIC) - MAGIC
    y = ang - jr * PIO2
    qi = jr.astype(jnp.int32)
    z = y * y
    sinp = ((S3 * z + S2) * z + S1) * z * y + y
    cosp = ((C3 * z + C2) * z + C1) * (z * z) - 0.5 * z + 1.0
    swap = (qi & 1) == 1
    nsin = (qi & 2) != 0
    ncos = ((qi + 1) & 2) != 0
    s = jnp.where(swap, cosp, sinp)
    s = jnp.where(nsin, -s, s)
    c = jnp.where(swap, sinp, cosp)
    c = jnp.where(ncos, -c, c)
    return s, c


def _make_sc_kernel(n, n_pad, n_edges):
    n_chunks_total = n_edges // CHUNK
    base_chunks = n_chunks_total // N_TILES
    extra = n_chunks_total - base_chunks * N_TILES
    npt = n_pad // N_SUBCORES

    mesh = plsc.VectorSubcoreMesh(core_axis_name="c", subcore_axis_name="s")

    @functools.partial(
        pl.kernel,
        out_type=(
            jax.ShapeDtypeStruct((N_CORES, 1, n_pad), jnp.float32),
            jax.ShapeDtypeStruct((N_CORES, 1, n_pad), jnp.float32),
        ),
        mesh=mesh,
        scratch_types=[
            pltpu.VMEM_SHARED((n_pad,), jnp.float32),
            pltpu.VMEM_SHARED((n_pad,), jnp.float32),
            pltpu.VMEM_SHARED((n_pad,), jnp.float32),
            pltpu.VMEM_SHARED((n_pad,), jnp.float32),
            pltpu.VMEM((ROWS_PER_CHUNK, ROW_W), jnp.int32),
            pltpu.VMEM((ROWS_PER_CHUNK, ROW_W), jnp.int32),
            pltpu.VMEM((8, CHUNK), jnp.float32),
            pltpu.VMEM((CHUNK,), jnp.float32),
            pltpu.VMEM((CHUNK,), jnp.float32),
            pltpu.VMEM((CHUNK,), jnp.float32),
            pltpu.VMEM((CHUNK,), jnp.float32),
            pltpu.VMEM((CHUNK,), jnp.float32),
            pltpu.VMEM((CHUNK,), jnp.float32),
            pltpu.VMEM((CHUNK,), jnp.float32),
            pltpu.VMEM((CHUNK,), jnp.float32),
            pltpu.VMEM((n_pad // N_SUBCORES,), jnp.float32),
            pltpu.SemaphoreType.DMA,
            pltpu.SemaphoreType.DMA,
            pltpu.SemaphoreType.DMA,
            pltpu.SemaphoreType.DMA,
        ],
    )
    def sc_kernel(x_hbm, ei_hbm, r_hbm, xx_hbm, gfr_hbm, bfr_hbm,
                  gto_hbm, bto_hbm, tap_hbm, shift_hbm, outp_hbm, outq_hbm,
                  va_sh, vm_sh, accp_sh, accq_sh, si, di, att,
                  g_vai, g_vmi, g_vaj, g_vmj, c_pf, c_qf, c_pt, c_qt,
                  vbuf, sem_in, sem_ga, sem_gb, sem_s):
        c = lax.axis_index("c")
        s = lax.axis_index("s")
        wid = s * N_CORES + c
        npt_ = n_pad // N_SUBCORES
        last = n - (N_SUBCORES - 1) * npt_

        fz16 = jnp.zeros((LANES,), jnp.float32)

        @pl.loop(0, npt_ // LANES)
        def _z(v):
            vbuf[pl.ds(v * LANES, LANES)] = fz16

        tsl = pl.ds(s * npt_, npt_)
        pltpu.sync_copy(vbuf, accp_sh.at[tsl])
        pltpu.sync_copy(vbuf, accq_sh.at[tsl])

        @pl.when(s < N_SUBCORES - 1)
        def _():
            pltpu.sync_copy(x_hbm.at[pl.ds(s * npt_, npt_)], vbuf)
            pltpu.sync_copy(vbuf, va_sh.at[tsl])
            pltpu.sync_copy(x_hbm.at[pl.ds(n + s * npt_, npt_)], vbuf)
            pltpu.sync_copy(vbuf, vm_sh.at[tsl])

        @pl.when(s == N_SUBCORES - 1)
        def _():
            lsl = pl.ds(0, last)
            ssl = pl.ds(s * npt_, last)
            pltpu.sync_copy(x_hbm.at[pl.ds(s * npt_, last)], vbuf.at[lsl])
            pltpu.sync_copy(vbuf.at[lsl], va_sh.at[ssl])
            pltpu.sync_copy(x_hbm.at[pl.ds(n + s * npt_, last)], vbuf.at[lsl])
            pltpu.sync_copy(vbuf.at[lsl], vm_sh.at[ssl])

        plsc.subcore_barrier()

        attr_hbm = (r_hbm, xx_hbm, gfr_hbm, bfr_hbm, gto_hbm, bto_hbm,
                    tap_hbm, shift_hbm)
        my_chunks = jnp.where(wid < extra, base_chunks + 1, base_chunks)

        @pl.loop(0, my_chunks)
        def _chunk(t):
            ki = wid + N_TILES * t
            eoff = ki * CHUNK
            roff = ki * ROWS_PER_CHUNK
            cps = [
                pltpu.async_copy(ei_hbm.at[0, pl.ds(roff, ROWS_PER_CHUNK)],
                                 si, sem_in),
                pltpu.async_copy(ei_hbm.at[1, pl.ds(roff, ROWS_PER_CHUNK)],
                                 di, sem_in),
            ]
            for m, h in enumerate(attr_hbm):
                cps.append(pltpu.async_copy(h.at[pl.ds(eoff, CHUNK)],
                                            att.at[m], sem_in))
            for cp in cps:
                cp.wait()

            def fire_row(j, sem):
                dsl = pl.ds(j * ROW_W, ROW_W)
                pltpu.async_copy(va_sh.at[si.at[j]], g_vai.at[dsl], sem)
                pltpu.async_copy(vm_sh.at[si.at[j]], g_vmi.at[dsl], sem)
                pltpu.async_copy(va_sh.at[di.at[j]], g_vaj.at[dsl], sem)
                pltpu.async_copy(vm_sh.at[di.at[j]], g_vmj.at[dsl], sem)

            def wait_row(j, sem):
                dsl = pl.ds(j * ROW_W, ROW_W)
                pltpu.make_async_copy(va_sh.at[si.at[j]], g_vai.at[dsl],
                                      sem).wait()
                pltpu.make_async_copy(vm_sh.at[si.at[j]], g_vmi.at[dsl],
                                      sem).wait()
                pltpu.make_async_copy(va_sh.at[di.at[j]], g_vaj.at[dsl],
                                      sem).wait()
                pltpu.make_async_copy(vm_sh.at[di.at[j]], g_vmj.at[dsl],
                                      sem).wait()

            def fire_scat(j):
                dsl = pl.ds(j * ROW_W, ROW_W)
                pltpu.async_copy(c_pf.at[dsl], accp_sh.at[si.at[j]], sem_s,
                                 add=True)
                pltpu.async_copy(c_qf.at[dsl], accq_sh.at[si.at[j]], sem_s,
                                 add=True)
                pltpu.async_copy(c_pt.at[dsl], accp_sh.at[di.at[j]], sem_s,
                                 add=True)
                pltpu.async_copy(c_qt.at[dsl], accq_sh.at[di.at[j]], sem_s,
                                 add=True)

            fire_row(0, sem_ga)

            @pl.loop(0, ROWS_PER_CHUNK)
            def _row(j):
                even = (j & 1) == 0

                @pl.when(even)
                def _():
                    @pl.when(j + 1 < ROWS_PER_CHUNK)
                    def _():
                        fire_row(j + 1, sem_gb)
                    wait_row(j, sem_ga)

                @pl.when(jnp.logical_not(even))
                def _():
                    @pl.when(j + 1 < ROWS_PER_CHUNK)
                    def _():
                        fire_row(j + 1, sem_ga)
                    wait_row(j, sem_gb)

                for v in range(ROW_W // LANES):
                    sl = pl.ds(j * ROW_W + v * LANES, LANES)
                    va_i = g_vai[sl]
                    vm_i = g_vmi[sl]
                    va_j = g_vaj[sl]
                    vm_j = g_vmj[sl]
                    br_r = att.at[0][sl]
                    br_x = att.at[1][sl]
                    gfr = att.at[2][sl]
                    bfr = att.at[3][sl]
                    gto = att.at[4][sl]
                    bto = att.at[5][sl]
                    tp = att.at[6][sl]
                    sh = att.at[7][sl]

                    inv_den = 1.0 / (br_r * br_r + br_x * br_x)
                    g_s = br_r * inv_den
                    b_s = -br_x * inv_den
                    inv_tap = 1.0 / tp
                    vit = vm_i * inv_tap
                    vit2 = vit * vit
                    vij = vit * vm_j
                    sin_f, cos_f = _sincos(va_i - va_j - sh)
                    gc = g_s * cos_f
                    gs_ = g_s * sin_f
                    bc = b_s * cos_f
                    bs_ = b_s * sin_f
                    c_pf[sl] = vit2 * (g_s + gfr) + vij * (-gc - bs_)
                    c_qf[sl] = -vit2 * (b_s + bfr) + vij * (-gs_ + bc)
                    vj2 = vm_j * vm_j
                    c_pt[sl] = vj2 * (g_s + gto) + vij * (-gc + bs_)
                    c_qt[sl] = -vj2 * (b_s + bto) + vij * (gs_ + bc)

                @pl.when(j > 0)
                def _():
                    fire_scat(j - 1)

            fire_scat(ROWS_PER_CHUNK - 1)

            @pl.loop(0, ROWS_PER_CHUNK)
            def _scw(j):
                dsl = pl.ds(j * ROW_W, ROW_W)
                pltpu.make_async_copy(c_pf.at[dsl], accp_sh.at[si.at[j]],
                                      sem_s).wait()
                pltpu.make_async_copy(c_qf.at[dsl], accq_sh.at[si.at[j]],
                                      sem_s).wait()
                pltpu.make_async_copy(c_pt.at[dsl], accp_sh.at[di.at[j]],
                                      sem_s).wait()
                pltpu.make_async_copy(c_qt.at[dsl], accq_sh.at[di.at[j]],
                                      sem_s).wait()

        plsc.subcore_barrier()

        dsl = pl.ds(s * npt, npt)
        pltpu.sync_copy(accp_sh.at[dsl], vbuf)
        pltpu.sync_copy(vbuf, outp_hbm.at[c, 0, dsl])
        pltpu.sync_copy(accq_sh.at[dsl], vbuf)
        pltpu.sync_copy(vbuf, outq_hbm.at[c, 0, dsl])

    return sc_kernel


def _finalize_body(pp_ref, qq_ref, x_ref, ps_ref, qs_ref, gs_ref, bs_ref,
                   bt_ref, vs_ref, o_ref):
    va = x_ref[0, :]
    vm = x_ref[1, :]
    vm2 = vm * vm
    p_calc = pp_ref[0, :] + pp_ref[1, :] + vm2 * gs_ref[...]
    q_calc = qq_ref[0, :] + qq_ref[1, :] - vm2 * bs_ref[...]
    bt = bt_ref[...]
    sl_mask = bt == 3
    pvsl = (bt == 2) | sl_mask
    o_ref[0, :] = jnp.where(sl_mask, va, ps_ref[...] - p_calc)
    o_ref[1, :] = jnp.where(pvsl, vm - vs_ref[...], qs_ref[...] - q_calc)


def kernel(x, edge_index, br_r, br_x, g_fr, b_fr, g_to, b_to, tap, shift,
           p_spec, q_spec, gs, bs, bus_type, vm_setpoint):
    n = x.shape[0] // 2
    e = edge_index.shape[1]
    group = N_SUBCORES * 128
    n_pad = ((n + group - 1) // group) * group
    ei3 = edge_index.reshape(2, e // ROW_W, ROW_W)

    partp, partq = _make_sc_kernel(n, n_pad, e)(
        x, ei3, br_r, br_x, g_fr, b_fr, g_to, b_to, tap, shift)
    pp = partp.reshape(2, n_pad)[:, :n]
    qq = partq.reshape(2, n_pad)[:, :n]

    out2 = pl.pallas_call(
        _finalize_body,
        out_shape=jax.ShapeDtypeStruct((2, n), jnp.float32),
    )(pp, qq, x.reshape(2, n), p_spec, q_spec, gs, bs, bus_type, vm_setpoint)
    return out2.reshape(2 * n)


# speedup vs baseline: 340.7234x; 1.0253x over previous
import functools

import jax
import jax.numpy as jnp
from jax import lax
from jax.experimental import pallas as pl
from jax.experimental.pallas import tpu as pltpu
from jax.experimental.pallas import tpu_sc as plsc

N_CORES = 2
N_SUBCORES = 16
LANES = 16
N_TILES = N_CORES * N_SUBCORES

ROW_W = 128
ROWS_PER_CHUNK = 40
CHUNK = ROWS_PER_CHUNK * ROW_W

TWO_OVER_PI = 0.6366197723675814
PIO2 = 1.5707963267948966
MAGIC = 12582912.0
S1, S2, S3 = -1.6666654611e-1, 8.3321608736e-3, -1.9515295891e-4
C1, C2, C3 = 4.166664568298827e-2, -1.388731625493765e-3, 2.443315711809948e-5


def _sincos(ang):
    jf = ang * TWO_OVER_PI
    jr = (jf + MAGIC) - MAGIC
    y = ang - jr * PIO2
    qi = jr.astype(jnp.int32)
    z = y * y
    sinp = ((S3 * z + S2) * z + S1) * z * y + y
    cosp = ((C3 * z + C2) * z + C1) * (z * z) - 0.5 * z + 1.0
    swap = (qi & 1) == 1
    nsin = (qi & 2) != 0
    ncos = ((qi + 1) & 2) != 0
    s = jnp.where(swap, cosp, sinp)
    s = jnp.where(nsin, -s, s)
    c = jnp.where(swap, sinp, cosp)
    c = jnp.where(ncos, -c, c)
    return s, c


def _make_sc_kernel(n, n_pad, n_edges):
    n_chunks_total = n_edges // CHUNK
    base_chunks = n_chunks_total // N_TILES
    extra = n_chunks_total - base_chunks * N_TILES
    npt = n_pad // N_SUBCORES

    mesh = plsc.VectorSubcoreMesh(core_axis_name="c", subcore_axis_name="s")

    @functools.partial(
        pl.kernel,
        out_type=(
            jax.ShapeDtypeStruct((N_CORES, 1, n_pad), jnp.float32),
            jax.ShapeDtypeStruct((N_CORES, 1, n_pad), jnp.float32),
        ),
        mesh=mesh,
        scratch_types=[
            pltpu.VMEM_SHARED((n_pad,), jnp.float32),
            pltpu.VMEM_SHARED((n_pad,), jnp.float32),
            pltpu.VMEM_SHARED((n_pad,), jnp.float32),
            pltpu.VMEM_SHARED((n_pad,), jnp.float32),
            pltpu.VMEM((2, ROWS_PER_CHUNK, ROW_W), jnp.int32),
            pltpu.VMEM((2, ROWS_PER_CHUNK, ROW_W), jnp.int32),
            pltpu.VMEM((8, CHUNK), jnp.float32),
            pltpu.VMEM((2, ROW_W), jnp.float32),
            pltpu.VMEM((2, ROW_W), jnp.float32),
            pltpu.VMEM((2, ROW_W), jnp.float32),
            pltpu.VMEM((2, ROW_W), jnp.float32),
            pltpu.VMEM((CHUNK,), jnp.float32),
            pltpu.VMEM((CHUNK,), jnp.float32),
            pltpu.VMEM((CHUNK,), jnp.float32),
            pltpu.VMEM((CHUNK,), jnp.float32),
            pltpu.VMEM((n_pad // N_SUBCORES,), jnp.float32),
            pltpu.SemaphoreType.DMA,
            pltpu.SemaphoreType.DMA,
            pltpu.SemaphoreType.DMA,
            pltpu.SemaphoreType.DMA,
        ],
    )
    def sc_kernel(x_hbm, ei_hbm, r_hbm, xx_hbm, gfr_hbm, bfr_hbm,
                  gto_hbm, bto_hbm, tap_hbm, shift_hbm, outp_hbm, outq_hbm,
                  va_sh, vm_sh, accp_sh, accq_sh, si, di, att,
                  g_vai, g_vmi, g_vaj, g_vmj, c_pf, c_qf, c_pt, c_qt,
                  vbuf, sem_in, sem_ga, sem_gb, sem_s):
        c = lax.axis_index("c")
        s = lax.axis_index("s")
        wid = s * N_CORES + c
        npt_ = n_pad // N_SUBCORES
        last = n - (N_SUBCORES - 1) * npt_

        fz16 = jnp.zeros((LANES,), jnp.float32)

        @pl.loop(0, npt_ // LANES)
        def _z(v):
            vbuf[pl.ds(v * LANES, LANES)] = fz16

        tsl = pl.ds(s * npt_, npt_)
        pltpu.sync_copy(vbuf, accp_sh.at[tsl])
        pltpu.sync_copy(vbuf, accq_sh.at[tsl])

        @pl.when(s < N_SUBCORES - 1)
        def _():
            pltpu.sync_copy(x_hbm.at[pl.ds(s * npt_, npt_)], vbuf)
            pltpu.sync_copy(vbuf, va_sh.at[tsl])
            pltpu.sync_copy(x_hbm.at[pl.ds(n + s * npt_, npt_)], vbuf)
            pltpu.sync_copy(vbuf, vm_sh.at[tsl])

        @pl.when(s == N_SUBCORES - 1)
        def _():
            lsl = pl.ds(0, last)
            ssl = pl.ds(s * npt_, last)
            pltpu.sync_copy(x_hbm.at[pl.ds(s * npt_, last)], vbuf.at[lsl])
            pltpu.sync_copy(vbuf.at[lsl], va_sh.at[ssl])
            pltpu.sync_copy(x_hbm.at[pl.ds(n + s * npt_, last)], vbuf.at[lsl])
            pltpu.sync_copy(vbuf.at[lsl], vm_sh.at[ssl])

        plsc.subcore_barrier()

        attr_hbm = (r_hbm, xx_hbm, gfr_hbm, bfr_hbm, gto_hbm, bto_hbm,
                    tap_hbm, shift_hbm)
        my_chunks = jnp.where(wid < extra, base_chunks + 1, base_chunks)

        def fire_linear(tt, pp):
            ki2 = wid + N_TILES * tt
            eoff = ki2 * CHUNK
            roff = ki2 * ROWS_PER_CHUNK
            pltpu.async_copy(ei_hbm.at[0, pl.ds(roff, ROWS_PER_CHUNK)],
                             si.at[pp], sem_in)
            pltpu.async_copy(ei_hbm.at[1, pl.ds(roff, ROWS_PER_CHUNK)],
                             di.at[pp], sem_in)
            for m, h in enumerate(attr_hbm):
                pltpu.async_copy(h.at[pl.ds(eoff, CHUNK)], att.at[m], sem_in)

        def wait_linear(tt, pp):
            ki2 = wid + N_TILES * tt
            eoff = ki2 * CHUNK
            roff = ki2 * ROWS_PER_CHUNK
            pltpu.make_async_copy(ei_hbm.at[0, pl.ds(roff, ROWS_PER_CHUNK)],
                                  si.at[pp], sem_in).wait()
            pltpu.make_async_copy(ei_hbm.at[1, pl.ds(roff, ROWS_PER_CHUNK)],
                                  di.at[pp], sem_in).wait()
            for m, h in enumerate(attr_hbm):
                pltpu.make_async_copy(h.at[pl.ds(eoff, CHUNK)], att.at[m],
                                      sem_in).wait()

        fire_linear(0, 0)

        @pl.loop(0, my_chunks)
        def _chunk(t):
            pp = t & 1
            wait_linear(t, pp)

            def fire_row(j, sem):
                jp = j & 1
                pltpu.async_copy(va_sh.at[si.at[pp, j]], g_vai.at[jp], sem)
                pltpu.async_copy(vm_sh.at[si.at[pp, j]], g_vmi.at[jp], sem)
                pltpu.async_copy(va_sh.at[di.at[pp, j]], g_vaj.at[jp], sem)
                pltpu.async_copy(vm_sh.at[di.at[pp, j]], g_vmj.at[jp], sem)

            def wait_row(j, sem):
                jp = j & 1
                pltpu.make_async_copy(va_sh.at[si.at[pp, j]], g_vai.at[jp],
                                      sem).wait()
                pltpu.make_async_copy(vm_sh.at[si.at[pp, j]], g_vmi.at[jp],
                                      sem).wait()
                pltpu.make_async_copy(va_sh.at[di.at[pp, j]], g_vaj.at[jp],
                                      sem).wait()
                pltpu.make_async_copy(vm_sh.at[di.at[pp, j]], g_vmj.at[jp],
                                      sem).wait()

            def fire_scat(j):
                dsl = pl.ds(j * ROW_W, ROW_W)
                pltpu.async_copy(c_pf.at[dsl], accp_sh.at[si.at[pp, j]],
                                 sem_s, add=True)
                pltpu.async_copy(c_qf.at[dsl], accq_sh.at[si.at[pp, j]],
                                 sem_s, add=True)
                pltpu.async_copy(c_pt.at[dsl], accp_sh.at[di.at[pp, j]],
                                 sem_s, add=True)
                pltpu.async_copy(c_qt.at[dsl], accq_sh.at[di.at[pp, j]],
                                 sem_s, add=True)

            fire_row(0, sem_ga)

            @pl.loop(0, ROWS_PER_CHUNK)
            def _row(j):
                even = (j & 1) == 0

                @pl.when(even)
                def _():
                    @pl.when(j + 1 < ROWS_PER_CHUNK)
                    def _():
                        fire_row(j + 1, sem_gb)
                    wait_row(j, sem_ga)

                @pl.when(jnp.logical_not(even))
                def _():
                    @pl.when(j + 1 < ROWS_PER_CHUNK)
                    def _():
                        fire_row(j + 1, sem_ga)
                    wait_row(j, sem_gb)

                jp = j & 1
                for v in range(ROW_W // LANES):
                    gl = pl.ds(v * LANES, LANES)
                    sl = pl.ds(j * ROW_W + v * LANES, LANES)
                    va_i = g_vai.at[jp][gl]
                    vm_i = g_vmi.at[jp][gl]
                    va_j = g_vaj.at[jp][gl]
                    vm_j = g_vmj.at[jp][gl]
                    br_r = att.at[0][sl]
                    br_x = att.at[1][sl]
                    gfr = att.at[2][sl]
                    bfr = att.at[3][sl]
                    gto = att.at[4][sl]
                    bto = att.at[5][sl]
                    tp = att.at[6][sl]
                    sh = att.at[7][sl]

                    inv_den = 1.0 / (br_r * br_r + br_x * br_x)
                    g_s = br_r * inv_den
                    b_s = -br_x * inv_den
                    inv_tap = 1.0 / tp
                    vit = vm_i * inv_tap
                    vit2 = vit * vit
                    vij = vit * vm_j
                    sin_f, cos_f = _sincos(va_i - va_j - sh)
                    gc = g_s * cos_f
                    gs_ = g_s * sin_f
                    bc = b_s * cos_f
                    bs_ = b_s * sin_f
                    c_pf[sl] = vit2 * (g_s + gfr) + vij * (-gc - bs_)
                    c_qf[sl] = -vit2 * (b_s + bfr) + vij * (-gs_ + bc)
                    vj2 = vm_j * vm_j
                    c_pt[sl] = vj2 * (g_s + gto) + vij * (-gc + bs_)
                    c_qt[sl] = -vj2 * (b_s + bto) + vij * (gs_ + bc)

                @pl.when(j > 0)
                def _():
                    fire_scat(j - 1)

            fire_scat(ROWS_PER_CHUNK - 1)

            @pl.when(t + 1 < my_chunks)
            def _():
                fire_linear(t + 1, 1 - pp)

            @pl.loop(0, ROWS_PER_CHUNK)
            def _scw(j):
                dsl = pl.ds(j * ROW_W, ROW_W)
                pltpu.make_async_copy(c_pf.at[dsl], accp_sh.at[si.at[pp, j]],
                                      sem_s).wait()
                pltpu.make_async_copy(c_qf.at[dsl], accq_sh.at[si.at[pp, j]],
                                      sem_s).wait()
                pltpu.make_async_copy(c_pt.at[dsl], accp_sh.at[di.at[pp, j]],
                                      sem_s).wait()
                pltpu.make_async_copy(c_qt.at[dsl], accq_sh.at[di.at[pp, j]],
                                      sem_s).wait()

        plsc.subcore_barrier()

        dsl = pl.ds(s * npt, npt)
        pltpu.sync_copy(accp_sh.at[dsl], vbuf)
        pltpu.sync_copy(vbuf, outp_hbm.at[c, 0, dsl])
        pltpu.sync_copy(accq_sh.at[dsl], vbuf)
        pltpu.sync_copy(vbuf, outq_hbm.at[c, 0, dsl])

    return sc_kernel


def _finalize_body(pp_ref, qq_ref, x_ref, ps_ref, qs_ref, gs_ref, bs_ref,
                   bt_ref, vs_ref, o_ref):
    va = x_ref[0, :]
    vm = x_ref[1, :]
    vm2 = vm * vm
    p_calc = pp_ref[0, :] + pp_ref[1, :] + vm2 * gs_ref[...]
    q_calc = qq_ref[0, :] + qq_ref[1, :] - vm2 * bs_ref[...]
    bt = bt_ref[...]
    sl_mask = bt == 3
    pvsl = (bt == 2) | sl_mask
    o_ref[0, :] = jnp.where(sl_mask, va, ps_ref[...] - p_calc)
    o_ref[1, :] = jnp.where(pvsl, vm - vs_ref[...], qs_ref[...] - q_calc)


def kernel(x, edge_index, br_r, br_x, g_fr, b_fr, g_to, b_to, tap, shift,
           p_spec, q_spec, gs, bs, bus_type, vm_setpoint):
    n = x.shape[0] // 2
    e = edge_index.shape[1]
    group = N_SUBCORES * 128
    n_pad = ((n + group - 1) // group) * group
    ei3 = edge_index.reshape(2, e // ROW_W, ROW_W)

    partp, partq = _make_sc_kernel(n, n_pad, e)(
        x, ei3, br_r, br_x, g_fr, b_fr, g_to, b_to, tap, shift)
    pp = partp.reshape(2, n_pad)[:, :n]
    qq = partq.reshape(2, n_pad)[:, :n]

    out2 = pl.pallas_call(
        _finalize_body,
        out_shape=jax.ShapeDtypeStruct((2, n), jnp.float32),
    )(pp, qq, x.reshape(2, n), p_spec, q_spec, gs, bs, bus_type, vm_setpoint)
    return out2.reshape(2 * n)
